# baseline (speedup 1.0000x reference)
import functools

import jax
import jax.numpy as jnp
from jax import lax
from jax.experimental import pallas as pl
from jax.experimental.pallas import tpu as pltpu
from jax.experimental.pallas import tpu_sc as plsc

S_DIM = 4096
T_DIM = 4096
C = 1048576

OUT_N = S_DIM * T_DIM
HALF = OUT_N // 2
DUMMY_PER_TILE = 32768
OUT_FULL = OUT_N + 32 * DUMMY_PER_TILE

NB = 8
BLK = 8192
CHUNK = NB * BLK
ZCH = 16384
ZN = HALF // 16 // ZCH


def _prep_body(w_ref, sig_ref, loss_ref):
    w = w_ref[...]
    sig = jax.nn.sigmoid(w)
    sig_ref[...] = sig
    ent = -(sig * jnp.log(sig + 1e-10) + (1.0 - sig) * jnp.log(1.0 - sig + 1e-10))
    part = jnp.sum(ent) * (1.0 / C)

    @pl.when(pl.program_id(0) == 0)
    def _():
        loss_ref[0, 0] = 0.0

    loss_ref[0, 0] += part


_prep = pl.pallas_call(
    _prep_body,
    grid=(8,),
    in_specs=[pl.BlockSpec((128, 1024), lambda i: (i, 0))],
    out_specs=[
        pl.BlockSpec((128, 1024), lambda i: (i, 0)),
        pl.BlockSpec((1, 1), lambda i: (0, 0), memory_space=pltpu.SMEM),
    ],
    out_shape=[
        jax.ShapeDtypeStruct((1024, 1024), jnp.float32),
        jax.ShapeDtypeStruct((1, 1), jnp.float32),
    ],
)


_mesh = plsc.VectorSubcoreMesh(core_axis_name="core", subcore_axis_name="sub")


@functools.partial(
    pl.kernel,
    mesh=_mesh,
    out_type=jax.ShapeDtypeStruct((OUT_FULL,), jnp.float32),
    scratch_types=[
        pltpu.VMEM((ZCH,), jnp.float32),
        pltpu.VMEM((BLK + 16,), jnp.int32),
        pltpu.VMEM((BLK,), jnp.float32),
        pltpu.VMEM((BLK,), jnp.int32),
        pltpu.VMEM((BLK,), jnp.float32),
        pltpu.SemaphoreType.DMA,
    ],
)
def _sc_scatter(a_hbm, v_hbm, out_hbm, zbuf, abuf, vbuf, fa, fv, sem):
    core = lax.axis_index("core")
    sub = lax.axis_index("sub")
    gtid = core * 16 + sub

    def zb(i, c):
        zbuf[pl.ds(i * 16, 16)] = jnp.zeros((16,), jnp.float32)
        return c

    lax.fori_loop(0, ZCH // 16, zb, 0)

    zbase = core * HALF + sub * (HALF // 16)

    def zfire(k, c):
        pltpu.make_async_copy(
            zbuf, out_hbm.at[pl.ds(zbase + k * ZCH, ZCH)], sem
        ).start()
        return c

    lax.fori_loop(0, ZN, zfire, 0)

    def zwait(k, c):
        pltpu.make_async_copy(
            zbuf, out_hbm.at[pl.ds(zbase + k * ZCH, ZCH)], sem
        ).wait()
        return c

    lax.fori_loop(0, ZN, zwait, 0)
    plsc.subcore_barrier()

    lo = core * HALF
    hi = lo + HALF
    dummy_base = OUT_N + gtid * DUMMY_PER_TILE + lax.iota(jnp.int32, 16)

    def blk_body(blk, c):
        base = sub * CHUNK + blk * BLK
        pltpu.sync_copy(a_hbm.at[pl.ds(base, BLK + 16)], abuf)
        pltpu.sync_copy(v_hbm.at[pl.ds(base, BLK)], vbuf)
        doff = (blk & 3) << 13

        def row_body(r, c2):
            k0 = r * 128
            for u in range(8):
                k = k0 + u * 16
                a = abuf[pl.ds(k, 16)]
                an = abuf[pl.ds(k + 1, 16)]
                v = vbuf[pl.ds(k, 16)]
                keep = (a >= lo) & (a < hi) & (a != an)
                kk = ((k & 63) << 7) | (k >> 6)
                dvec = dummy_base + (doff + kk)
                fa[pl.ds(k, 16)] = jnp.where(keep, a, dvec)
                fv[pl.ds(k, 16)] = v
            return c2

        lax.fori_loop(0, 64, row_body, 0)
        pltpu.async_copy(fv, out_hbm.at[fa], sem)
        pltpu.make_async_copy(fv, out_hbm.at[fa], sem).wait()
        return c

    lax.fori_loop(0, NB, blk_body, 0)


def kernel(weight, s_idx, t_idx, S, T):
    w2 = weight.reshape(1024, 1024)
    sig, loss = _prep(w2)
    addr = s_idx * T_DIM + t_idx
    a_s, v_s = lax.sort((addr, sig.reshape(C)), dimension=0,
                        is_stable=False, num_keys=1)
    a_pad = jnp.concatenate([a_s, jnp.full((16,), -1, jnp.int32)])
    flat = _sc_scatter(a_pad, v_s)
    return flat[:OUT_N].reshape(S_DIM, T_DIM), loss.reshape(())


# speedup vs baseline: 7.1189x; 7.1189x over previous
import functools

import jax
import jax.numpy as jnp
from jax import lax
from jax.experimental import pallas as pl
from jax.experimental.pallas import tpu as pltpu
from jax.experimental.pallas import tpu_sc as plsc

S_DIM = 4096
T_DIM = 4096
C = 1048576

OUT_N = S_DIM * T_DIM
WIN = 1048576
NWIN = OUT_N // WIN
PASSES = NWIN // 2
PAD = 16512
STG = 8192
ZCH = 16384


def _prep_body(w_ref, sig_ref, loss_ref):
    w = w_ref[...]
    sig = jax.nn.sigmoid(w)
    sig_ref[...] = sig
    ent = -(sig * jnp.log(sig + 1e-10) + (1.0 - sig) * jnp.log(1.0 - sig + 1e-10))
    part = jnp.sum(ent) * (1.0 / C)

    @pl.when(pl.program_id(0) == 0)
    def _():
        loss_ref[0, 0] = 0.0

    loss_ref[0, 0] += part


_prep = pl.pallas_call(
    _prep_body,
    grid=(8,),
    in_specs=[pl.BlockSpec((128, 1024), lambda i: (i, 0))],
    out_specs=[
        pl.BlockSpec((128, 1024), lambda i: (i, 0)),
        pl.BlockSpec((1, 1), lambda i: (0, 0), memory_space=pltpu.SMEM),
    ],
    out_shape=[
        jax.ShapeDtypeStruct((1024, 1024), jnp.float32),
        jax.ShapeDtypeStruct((1, 1), jnp.float32),
    ],
)


_mesh = plsc.VectorSubcoreMesh(core_axis_name="core", subcore_axis_name="sub")


@functools.partial(
    pl.kernel,
    mesh=_mesh,
    out_type=jax.ShapeDtypeStruct((OUT_N,), jnp.float32),
    scratch_types=[
        pltpu.VMEM_SHARED((WIN + 128,), jnp.float32),
        pltpu.VMEM((ZCH,), jnp.float32),
        pltpu.VMEM((32,), jnp.int32),
        pltpu.VMEM((STG + 16,), jnp.int32),
        pltpu.VMEM((STG,), jnp.float32),
        pltpu.VMEM((STG,), jnp.int32),
        pltpu.SemaphoreType.DMA,
    ],
)
def _sc_scatter(a_hbm, v_hbm, bnd_hbm, out_hbm, shared, zbuf, bbuf, abuf,
                vbuf, ibuf, sem):
    core = lax.axis_index("core")
    sub = lax.axis_index("sub")

    def zb(i, c):
        zbuf[pl.ds(i * 16, 16)] = jnp.zeros((16,), jnp.float32)
        return c

    lax.fori_loop(0, ZCH // 16, zb, 0)
    pltpu.sync_copy(bnd_hbm, bbuf)
    blovec = bbuf[pl.ds(0, 16)]
    bhivec = bbuf[pl.ds(16, 16)]
    iota = lax.iota(jnp.int32, 16)
    tslice = WIN // 16

    def one_pass(widx):
        wlo = widx * WIN

        def zfire(j, c2):
            pltpu.make_async_copy(
                zbuf, shared.at[pl.ds(sub * tslice + j * ZCH, ZCH)], sem
            ).start()
            return c2

        lax.fori_loop(0, tslice // ZCH, zfire, 0)

        def zwait(j, c2):
            pltpu.make_async_copy(
                zbuf, shared.at[pl.ds(sub * tslice + j * ZCH, ZCH)], sem
            ).wait()
            return c2

        lax.fori_loop(0, tslice // ZCH, zwait, 0)
        plsc.subcore_barrier()

        b_lo = blovec[widx]
        b_hi = bhivec[widx]
        share = (b_hi - b_lo + 15) >> 4
        start = b_lo + sub * share
        end = jnp.minimum(start + share, b_hi)
        trips = (share + 8 + STG - 1) >> 13

        def trip(t, c2):
            sbase = pl.multiple_of((start + t * STG) & ~7, 8)
            pltpu.sync_copy(a_hbm.at[pl.ds(sbase, STG + 16)], abuf)
            pltpu.sync_copy(v_hbm.at[pl.ds(sbase, STG)], vbuf)

            def grp(g, c3):
                k = g * 16
                a = abuf[pl.ds(k, 16)]
                an = abuf[pl.ds(k + 1, 16)]
                pos = sbase + k + iota
                keep = (a != an) & (pos >= start) & (pos < end)
                dvec = WIN + (k & 0x70) + iota
                ibuf[pl.ds(k, 16)] = jnp.where(keep, a - wlo, dvec)
                return c3

            lax.fori_loop(0, STG // 16, grp, 0)
            pltpu.sync_copy(vbuf, shared.at[ibuf], add=True)
            return c2

        lax.fori_loop(0, trips, trip, 0)
        plsc.subcore_barrier()

        pltpu.sync_copy(
            shared.at[pl.ds(sub * tslice, tslice)],
            out_hbm.at[pl.ds(wlo + sub * tslice, tslice)],
        )
        plsc.subcore_barrier()

    for p in range(PASSES):
        for cc in range(2):
            @pl.when(core == cc)
            def _(p=p, cc=cc):
                one_pass(cc * PASSES + p)


def kernel(weight, s_idx, t_idx, S, T):
    w2 = weight.reshape(1024, 1024)
    sig, loss = _prep(w2)
    addr = s_idx * T_DIM + t_idx
    a_s, v_s = lax.sort((addr, sig.reshape(C)), dimension=0,
                        is_stable=False, num_keys=1)
    bnd = jnp.searchsorted(a_s, jnp.arange(NWIN + 1, dtype=jnp.int32) * WIN
                           ).astype(jnp.int32)
    bnd = jnp.concatenate([bnd[:16], bnd[1:17]])
    a_pad = jnp.concatenate([a_s, jnp.zeros((PAD,), jnp.int32)])
    v_pad = jnp.concatenate([v_s, jnp.zeros((PAD,), jnp.float32)])
    flat = _sc_scatter(a_pad, v_pad, bnd)
    return flat.reshape(S_DIM, T_DIM), loss.reshape(())
